# Initial kernel scaffold; baseline (speedup 1.0000x reference)
import functools

import jax
import jax.numpy as jnp
from jax import lax
from jax.experimental import pallas as pl
from jax.experimental.pallas import tpu as pltpu
from jax.experimental.pallas import tpu_sc as plsc

N_NODES = 10000
N_EDGES = 320000
D = 128
NPAD = 10016
LW = 16
CH = 100
NC = 2
NS = 16
E_PER_TILE = N_EDGES // (NC * NS)
NCHUNK = E_PER_TILE // CH
ROWS_PER_TILE = N_NODES // NS
DPAD_ROWS = NPAD // NS


def _deg_body(src_h, dst_h, out_h, idx_s, idx_d, ones_v, zbuf, dsrc_sh,
              ddst_sh, sem):
    cid = lax.axis_index("c")
    sid = lax.axis_index("s")

    @pl.loop(0, DPAD_ROWS)
    def _(i):
        zbuf[i, :] = jnp.zeros((LW,), jnp.float32)

    @pl.loop(0, CH)
    def _(i):
        ones_v[i, :] = jnp.ones((LW,), jnp.float32)

    pltpu.sync_copy(zbuf, dsrc_sh.at[pl.ds(sid * DPAD_ROWS, DPAD_ROWS)])
    pltpu.sync_copy(zbuf, ddst_sh.at[pl.ds(sid * DPAD_ROWS, DPAD_ROWS)])
    plsc.subcore_barrier()

    rowbase = (cid * NS + sid) * NCHUNK
    pltpu.sync_copy(src_h.at[pl.ds(rowbase, NCHUNK)], idx_s)
    pltpu.sync_copy(dst_h.at[pl.ds(rowbase, NCHUNK)], idx_d)

    @pl.loop(0, NCHUNK // 4)
    def _(i):
        descs = []
        for b in range(4):
            g = i * 4 + b
            descs.append(pltpu.async_copy(ones_v, dsrc_sh.at[idx_s.at[g]],
                                          sem, add=True))
            descs.append(pltpu.async_copy(ones_v, ddst_sh.at[idx_d.at[g]],
                                          sem, add=True))
        for d in descs:
            d.wait()

    plsc.subcore_barrier()

    r0 = sid * DPAD_ROWS
    pltpu.sync_copy(dsrc_sh.at[pl.ds(r0, DPAD_ROWS)],
                    out_h.at[cid, 0, pl.ds(r0, DPAD_ROWS)])
    pltpu.sync_copy(ddst_sh.at[pl.ds(r0, DPAD_ROWS)],
                    out_h.at[cid, 1, pl.ds(r0, DPAD_ROWS)])


def _agg_body(hp_h, src_h, dst_h, out_h, idx_s, idx_d, r0_v, r1_v, r2_v,
              r3_v, zbuf, agg_sh, sem0, sem1, sem2, sem3):
    cid = lax.axis_index("c")
    sid = lax.axis_index("s")
    rows = (r0_v, r1_v, r2_v, r3_v)
    sems = (sem0, sem1, sem2, sem3)

    @pl.loop(0, 125)
    def _(i):
        for c in range(D // LW):
            zbuf[i, pl.ds(c * LW, LW)] = jnp.zeros((LW,), jnp.float32)

    for j in range(ROWS_PER_TILE // 125):
        pltpu.sync_copy(zbuf,
                        agg_sh.at[pl.ds(sid * ROWS_PER_TILE + j * 125, 125)])
    plsc.subcore_barrier()

    rowbase = (cid * NS + sid) * NCHUNK
    pltpu.sync_copy(src_h.at[pl.ds(rowbase, NCHUNK)], idx_s)
    pltpu.sync_copy(dst_h.at[pl.ds(rowbase, NCHUNK)], idx_d)

    for b in range(4):
        pltpu.async_copy(hp_h.at[idx_s.at[b]], rows[b], sems[b])

    def _step(g, b, fire_next):
        pltpu.make_async_copy(hp_h.at[idx_s.at[g]], rows[b], sems[b]).wait()
        pltpu.sync_copy(rows[b], agg_sh.at[idx_d.at[g]], add=True)
        if fire_next:
            pltpu.async_copy(hp_h.at[idx_s.at[g + 4]], rows[b], sems[b])

    @pl.loop(0, NCHUNK // 4 - 1)
    def _(i):
        for b in range(4):
            _step(i * 4 + b, b, True)

    for b in range(4):
        _step(NCHUNK - 4 + b, b, False)

    plsc.subcore_barrier()

    for j in range(ROWS_PER_TILE // 125):
        r0 = sid * ROWS_PER_TILE + j * 125
        pltpu.sync_copy(agg_sh.at[pl.ds(r0, 125)],
                        out_h.at[cid, pl.ds(r0, 125)])


def _make_sc_kernels():
    mesh = plsc.VectorSubcoreMesh(core_axis_name="c", subcore_axis_name="s")
    assert mesh.num_cores == NC and mesh.num_subcores == NS

    deg = functools.partial(
        pl.kernel,
        mesh=mesh,
        out_type=jax.ShapeDtypeStruct((NC, 2, NPAD, LW), jnp.float32),
        scratch_types=[
            pltpu.VMEM((NCHUNK, CH), jnp.int32),
            pltpu.VMEM((NCHUNK, CH), jnp.int32),
            pltpu.VMEM((CH, LW), jnp.float32),
            pltpu.VMEM((DPAD_ROWS, LW), jnp.float32),
            pltpu.VMEM_SHARED((NPAD, LW), jnp.float32),
            pltpu.VMEM_SHARED((NPAD, LW), jnp.float32),
            pltpu.SemaphoreType.DMA,
        ],
    )(_deg_body)

    agg = functools.partial(
        pl.kernel,
        mesh=mesh,
        out_type=jax.ShapeDtypeStruct((NC, N_NODES, D), jnp.float32),
        scratch_types=[
            pltpu.VMEM((NCHUNK, CH), jnp.int32),
            pltpu.VMEM((NCHUNK, CH), jnp.int32),
            pltpu.VMEM((CH, D), jnp.float32),
            pltpu.VMEM((CH, D), jnp.float32),
            pltpu.VMEM((CH, D), jnp.float32),
            pltpu.VMEM((CH, D), jnp.float32),
            pltpu.VMEM((125, D), jnp.float32),
            pltpu.VMEM_SHARED((N_NODES, D), jnp.float32),
            pltpu.SemaphoreType.DMA,
            pltpu.SemaphoreType.DMA,
            pltpu.SemaphoreType.DMA,
            pltpu.SemaphoreType.DMA,
        ],
    )(_agg_body)
    return deg, agg


def _prep_body(x_ref, w_ref, degp_ref, hp_ref, nsrc_ref, ndst_ref):
    deg_s = degp_ref[0, 0] + degp_ref[1, 0]
    deg_d = degp_ref[0, 1] + degp_ref[1, 1]
    nsrc = lax.rsqrt(jnp.maximum(deg_s, 1.0))
    ndst = lax.rsqrt(jnp.maximum(deg_d, 1.0))
    nsrc_ref[...] = nsrc
    ndst_ref[...] = ndst
    h = jnp.dot(x_ref[...], w_ref[...], preferred_element_type=jnp.float32)
    hp_ref[...] = h * nsrc[:N_NODES, 0:1]


def _layer_tail(aggp_ref, ndst_ref, b_ref, g_ref, be_ref, a_ref):
    y = (aggp_ref[0] + aggp_ref[1]) * ndst_ref[0:N_NODES, 0:1] + b_ref[...]
    m = jnp.mean(y, axis=0, keepdims=True)
    v = jnp.mean((y - m) * (y - m), axis=0, keepdims=True)
    yn = g_ref[...] * (y - m) * lax.rsqrt(v + 1e-5) + be_ref[...]
    return jnp.where(yn >= 0, yn, a_ref[0, 0] * yn)


def _mid_body(aggp_ref, ndst_ref, b_ref, g_ref, be_ref, a_ref, w_ref,
              nsrc_ref, hp_ref):
    h = _layer_tail(aggp_ref, ndst_ref, b_ref, g_ref, be_ref, a_ref)
    h2 = jnp.dot(h, w_ref[...], preferred_element_type=jnp.float32)
    hp_ref[...] = h2 * nsrc_ref[0:N_NODES, 0:1]


def _final_body(aggp_ref, ndst_ref, b_ref, g_ref, be_ref, a_ref, out_ref):
    out_ref[...] = _layer_tail(aggp_ref, ndst_ref, b_ref, g_ref, be_ref,
                               a_ref)


def kernel(clm_all, inputs, W1, b1, gamma1, beta1, a1, W2, b2, gamma2, beta2,
           a2):
    src2 = clm_all[0].reshape(N_EDGES // CH, CH)
    dst2 = clm_all[1].reshape(N_EDGES // CH, CH)

    deg_k, agg_k = _make_sc_kernels()

    degp = deg_k(src2, dst2)

    hp1, nsrc, ndst = pl.pallas_call(
        _prep_body,
        out_shape=[
            jax.ShapeDtypeStruct((N_NODES, D), jnp.float32),
            jax.ShapeDtypeStruct((NPAD, LW), jnp.float32),
            jax.ShapeDtypeStruct((NPAD, LW), jnp.float32),
        ],
    )(inputs, W1, degp)

    aggp1 = agg_k(hp1, src2, dst2)

    hp2 = pl.pallas_call(
        _mid_body,
        out_shape=jax.ShapeDtypeStruct((N_NODES, D), jnp.float32),
    )(aggp1, ndst, b1.reshape(1, D), gamma1.reshape(1, D),
      beta1.reshape(1, D), a1.reshape(1, 1), W2, nsrc)

    aggp2 = agg_k(hp2, src2, dst2)

    out = pl.pallas_call(
        _final_body,
        out_shape=jax.ShapeDtypeStruct((N_NODES, D), jnp.float32),
    )(aggp2, ndst, b2.reshape(1, D), gamma2.reshape(1, D),
      beta2.reshape(1, D), a2.reshape(1, 1))

    return out


# speedup vs baseline: 5.3964x; 5.3964x over previous
import functools

import jax
import jax.numpy as jnp
from jax import lax
from jax.experimental import pallas as pl
from jax.experimental.pallas import tpu as pltpu
from jax.experimental.pallas import tpu_sc as plsc

N_NODES = 10000
N_EDGES = 320000
D = 128
LW = 16
NC = 2
NS = 16
CH = 128
EPAD = 323584
PADNODE = 10100
NCHD = EPAD // (NC * NS * CH)
NCHA = EPAD // (NS * CH)
NPAD = 10112
ROWS_PER_TILE = NPAD // NS
HALF = NPAD // NC
ACC = 5120
TRASH = HALF
AROWS_PER_TILE = ACC // NS


def _deg_body(src_h, dst_h, zin_h, out_h, idx_s, idx_d, ones_v, hist_sh):
    cid = lax.axis_index("c")
    sid = lax.axis_index("s")

    @pl.loop(0, CH)
    def _(i):
        for c in range(D // LW):
            ones_v[i, pl.ds(c * LW, LW)] = jnp.ones((LW,), jnp.float32)

    wid = cid * NS + sid
    pltpu.sync_copy(src_h.at[wid], idx_s)
    pltpu.sync_copy(dst_h.at[wid], idx_d)

    r0 = sid * ROWS_PER_TILE
    for phase, idx in ((0, idx_s), (1, idx_d)):
        pltpu.sync_copy(zin_h, hist_sh.at[pl.ds(r0, ROWS_PER_TILE)])
        plsc.subcore_barrier()

        @pl.loop(0, NCHD)
        def _(g):
            pltpu.sync_copy(ones_v, hist_sh.at[idx.at[g]], add=True)

        plsc.subcore_barrier()
        pltpu.sync_copy(hist_sh.at[pl.ds(r0, ROWS_PER_TILE)],
                        out_h.at[cid, phase, pl.ds(r0, ROWS_PER_TILE)])
        plsc.subcore_barrier()


def _agg_body(hp_h, src_h, dst_h, zin_h, out_h, idx_s, idx_d, r0_v, r1_v,
              agg_sh, sem0, sem1):
    cid = lax.axis_index("c")
    sid = lax.axis_index("s")
    rows = (r0_v, r1_v)
    sems = (sem0, sem1)

    pltpu.sync_copy(zin_h, agg_sh.at[pl.ds(sid * AROWS_PER_TILE,
                                           AROWS_PER_TILE)])
    plsc.subcore_barrier()

    pltpu.sync_copy(src_h.at[sid], idx_s)
    pltpu.sync_copy(dst_h.at[cid, sid], idx_d)

    for b in range(2):
        pltpu.async_copy(hp_h.at[idx_s.at[b]], rows[b], sems[b])

    def _step(g, b, fire_next):
        pltpu.make_async_copy(hp_h.at[idx_s.at[g]], rows[b], sems[b]).wait()
        pltpu.sync_copy(rows[b], agg_sh.at[idx_d.at[g]], add=True)
        if fire_next:
            pltpu.async_copy(hp_h.at[idx_s.at[g + 2]], rows[b], sems[b])

    @pl.loop(0, NCHA // 2 - 1)
    def _(i):
        for b in range(2):
            _step(i * 2 + b, b, True)

    for b in range(2):
        _step(NCHA - 2 + b, b, False)

    plsc.subcore_barrier()

    r0 = sid * AROWS_PER_TILE
    pltpu.sync_copy(agg_sh.at[pl.ds(r0, AROWS_PER_TILE)],
                    out_h.at[cid, pl.ds(r0, AROWS_PER_TILE)])


def _make_sc_kernels():
    mesh = plsc.VectorSubcoreMesh(core_axis_name="c", subcore_axis_name="s")
    assert mesh.num_cores == NC and mesh.num_subcores == NS

    deg = functools.partial(
        pl.kernel,
        mesh=mesh,
        out_type=jax.ShapeDtypeStruct((NC, 2, NPAD, D), jnp.float32),
        scratch_types=[
            pltpu.VMEM((NCHD, CH), jnp.int32),
            pltpu.VMEM((NCHD, CH), jnp.int32),
            pltpu.VMEM((CH, D), jnp.float32),
            pltpu.VMEM_SHARED((NPAD, D), jnp.float32),
        ],
    )(_deg_body)

    agg = functools.partial(
        pl.kernel,
        mesh=mesh,
        out_type=jax.ShapeDtypeStruct((NC, ACC, D), jnp.float32),
        scratch_types=[
            pltpu.VMEM((NCHA, CH), jnp.int32),
            pltpu.VMEM((NCHA, CH), jnp.int32),
            pltpu.VMEM((CH, D), jnp.float32),
            pltpu.VMEM((CH, D), jnp.float32),
            pltpu.VMEM_SHARED((ACC, D), jnp.float32),
            pltpu.SemaphoreType.DMA,
            pltpu.SemaphoreType.DMA,
        ],
    )(_agg_body)
    return deg, agg


def _prep_body(x_ref, w_ref, degp_ref, dst3_ref, hp_ref, nsrc_ref, ndst_ref,
               dstloc_ref):
    deg_s = degp_ref[0, 0, :, 0:1] + degp_ref[1, 0, :, 0:1]
    deg_d = degp_ref[0, 1, :, 0:1] + degp_ref[1, 1, :, 0:1]
    nsrc = lax.rsqrt(jnp.maximum(deg_s, 1.0))
    ndst = lax.rsqrt(jnp.maximum(deg_d, 1.0))
    nsrc_ref[...] = nsrc
    ndst_ref[...] = ndst
    d3 = dst3_ref[...]
    for c in range(NC):
        loc = d3 - c * HALF
        ok = (loc >= 0) & (loc < HALF)
        dstloc_ref[c] = jnp.where(ok, loc, TRASH)
    h = jnp.dot(x_ref[...], w_ref[...], preferred_element_type=jnp.float32)
    hp_ref[0:N_NODES, :] = h * nsrc[:N_NODES, 0:1]
    hp_ref[N_NODES:NPAD, :] = jnp.zeros((NPAD - N_NODES, D), jnp.float32)


def _layer_tail(aggp_ref, ndst_ref, b_ref, g_ref, be_ref, a_ref):
    agg = jnp.concatenate(
        [aggp_ref[0, 0:HALF], aggp_ref[1, 0:N_NODES - HALF]], axis=0)
    y = agg * ndst_ref[0:N_NODES, 0:1] + b_ref[...]
    m = jnp.mean(y, axis=0, keepdims=True)
    v = jnp.mean((y - m) * (y - m), axis=0, keepdims=True)
    yn = g_ref[...] * (y - m) * lax.rsqrt(v + 1e-5) + be_ref[...]
    return jnp.where(yn >= 0, yn, a_ref[0, 0] * yn)


def _mid_body(aggp_ref, ndst_ref, b_ref, g_ref, be_ref, a_ref, w_ref,
              nsrc_ref, hp_ref):
    h = _layer_tail(aggp_ref, ndst_ref, b_ref, g_ref, be_ref, a_ref)
    h2 = jnp.dot(h, w_ref[...], preferred_element_type=jnp.float32)
    hp_ref[0:N_NODES, :] = h2 * nsrc_ref[0:N_NODES, 0:1]
    hp_ref[N_NODES:NPAD, :] = jnp.zeros((NPAD - N_NODES, D), jnp.float32)


def _final_body(aggp_ref, ndst_ref, b_ref, g_ref, be_ref, a_ref, out_ref):
    out_ref[...] = _layer_tail(aggp_ref, ndst_ref, b_ref, g_ref, be_ref,
                               a_ref)


def kernel(clm_all, inputs, W1, b1, gamma1, beta1, a1, W2, b2, gamma2, beta2,
           a2):
    pad = jnp.full((EPAD - N_EDGES,), PADNODE, jnp.int32)
    srcp = jnp.concatenate([clm_all[0], pad])
    dstp = jnp.concatenate([clm_all[1], pad])
    src2 = srcp.reshape(NC * NS, NCHD, CH)
    dst2 = dstp.reshape(NC * NS, NCHD, CH)
    src3 = srcp.reshape(NS, NCHA, CH)
    dst3 = dstp.reshape(NS, NCHA, CH)
    zindeg = jnp.zeros((ROWS_PER_TILE, D), jnp.float32)
    zin128 = jnp.zeros((AROWS_PER_TILE, D), jnp.float32)

    deg_k, agg_k = _make_sc_kernels()

    degp = deg_k(src2, dst2, zindeg)

    hp1, nsrc, ndst, dstloc = pl.pallas_call(
        _prep_body,
        out_shape=[
            jax.ShapeDtypeStruct((NPAD, D), jnp.float32),
            jax.ShapeDtypeStruct((NPAD, 1), jnp.float32),
            jax.ShapeDtypeStruct((NPAD, 1), jnp.float32),
            jax.ShapeDtypeStruct((NC, NS, NCHA, CH), jnp.int32),
        ],
    )(inputs, W1, degp, dst3)

    aggp1 = agg_k(hp1, src3, dstloc, zin128)

    hp2 = pl.pallas_call(
        _mid_body,
        out_shape=jax.ShapeDtypeStruct((NPAD, D), jnp.float32),
    )(aggp1, ndst, b1.reshape(1, D), gamma1.reshape(1, D),
      beta1.reshape(1, D), a1.reshape(1, 1), W2, nsrc)

    aggp2 = agg_k(hp2, src3, dstloc, zin128)

    out = pl.pallas_call(
        _final_body,
        out_shape=jax.ShapeDtypeStruct((N_NODES, D), jnp.float32),
    )(aggp2, ndst, b2.reshape(1, D), gamma2.reshape(1, D),
      beta2.reshape(1, D), a2.reshape(1, 1))

    return out
